# baseline (speedup 1.0000x reference)
import functools

import jax
import jax.numpy as jnp
from jax import lax
from jax.experimental import pallas as pl
from jax.experimental.pallas import tpu as pltpu
from jax.experimental.pallas import tpu_sc as plsc

N = 44000
E = 704000
HID = 128
GROUP = 22
ROWS = N // GROUP

NP = 44032
NBLK = 5504
EP = NBLK * 128
NC, NS = 2, 16
NW = NC * NS
BPT = NBLK // NW
CH = 4
NCHUNK = BPT // CH

NPART = 8
PS = NP // NPART
SLAB = PS
DUMP = 0
SPT = SLAB // NS
OPT = PS // NS

GREC = 384
GCAPG = 260
GCAPW = GCAPG * GREC
BKT_WORDS = NW * NPART * GCAPW
PCAP = 800

_MESH = dict(core_axis_name="c", subcore_axis_name="s", num_cores=NC,
             num_subcores=NS)


def _leaky(v):
    return jnp.where(v >= 0, v, 0.01 * v)


def _worker(c, s):
    return c * NS + s


def _p1_body(colb, rowb, ewb, deg, bkt, cnts, colv, rowv, ewv, pcol, prow,
             pew, pstage, cntv, zv, degsp):
    c = lax.axis_index("c")
    s = lax.axis_index("s")
    w = _worker(c, s)
    zw = NP // NS
    iota16 = lax.iota(jnp.int32, 16)

    def zfill(i, _):
        zv[pl.ds(i * 16, 16)] = jnp.zeros((16,), jnp.float32)
        return 0

    lax.fori_loop(0, zw // 16, zfill, 0)
    pltpu.sync_copy(zv, degsp.at[pl.ds(s * zw, zw)])
    plsc.subcore_barrier()

    base = w * BPT

    def _flush(p):
        def fbody(j, fp):
            off = p * PCAP + j * 128
            for t in range(8):
                pstage[pl.ds(t * 16, 16)] = pcol[pl.ds(off + t * 16, 16)]
                pstage[pl.ds(128 + t * 16, 16)] = prow[pl.ds(off + t * 16,
                                                             16)]
                pstage[pl.ds(256 + t * 16, 16)] = (
                    plsc.bitcast(pew[pl.ds(off + t * 16, 16)], jnp.int32))
            addr = (w * NPART + p) * GCAPW + fp * GREC
            pltpu.sync_copy(pstage.at[pl.ds(0, GREC)],
                            bkt.at[pl.ds(addr, GREC)])
            return fp + 1

        return fbody

    def chunk(i, carry):
        cnt = [jnp.int32(0)] * NPART
        fs = list(carry)
        cb = base + i * CH
        pltpu.sync_copy(colb.at[pl.ds(cb, CH)], colv)
        pltpu.sync_copy(rowb.at[pl.ds(cb, CH)], rowv)
        pltpu.sync_copy(ewb.at[pl.ds(cb, CH)], ewv)
        for j in range(CH):
            pltpu.sync_copy(ewv.at[j], degsp.at[colv.at[j]], add=True)
        for g in range(CH * 8):
            jj, gg = divmod(g, 8)
            c16 = colv[jj, pl.ds(gg * 16, 16)]
            r16 = rowv[jj, pl.ds(gg * 16, 16)]
            e16 = ewv[jj, pl.ds(gg * 16, 16)]
            ge = [None] + [c16 >= (p * PS) for p in range(1, NPART)]
            for p in range(NPART):
                if p == 0:
                    m = jnp.logical_not(ge[1])
                elif p == NPART - 1:
                    m = ge[p]
                else:
                    m = jnp.logical_xor(ge[p], ge[p + 1])
                pbase = p * PCAP
                plsc.store_compressed(pcol.at[pl.ds(pbase + cnt[p], 16)],
                                      c16 - p * PS, mask=m)
                plsc.store_compressed(prow.at[pl.ds(pbase + cnt[p], 16)], r16,
                                      mask=m)
                plsc.store_compressed(pew.at[pl.ds(pbase + cnt[p], 16)], e16,
                                      mask=m)
                cnt[p] = cnt[p] + jnp.sum(m.astype(jnp.int32))
        for p in range(NPART):
            cntp = cnt[p]
            for t in range(8):
                pcol[pl.ds(p * PCAP + cntp + t * 16, 16)] = jnp.full(
                    (16,), DUMP, jnp.int32)
                prow[pl.ds(p * PCAP + cntp + t * 16, 16)] = jnp.zeros(
                    (16,), jnp.int32)
                pew[pl.ds(p * PCAP + cntp + t * 16, 16)] = jnp.zeros(
                    (16,), jnp.float32)
            fs[p] = lax.fori_loop(0, (cntp + 127) // 128, _flush(p), fs[p])
        return tuple(fs)

    carry = lax.fori_loop(0, NCHUNK, chunk, (jnp.int32(0),) * NPART)

    cv = jnp.zeros((16,), jnp.int32)
    for p in range(NPART):
        cv = jnp.where(iota16 == p, jnp.broadcast_to(carry[p], (16,)), cv)

    cntv[pl.ds(0, 16)] = cv
    pltpu.sync_copy(cntv, cnts.at[pl.ds(w * 16, 16)])

    plsc.subcore_barrier()
    pltpu.sync_copy(degsp.at[pl.ds(s * zw, zw)], zv)
    pltpu.sync_copy(zv, deg.at[pl.ds(c * NP + s * zw, zw)])


def _p1(colb, rowb, ewb):
    zr = (NP // NS) // 16
    return pl.kernel(
        _p1_body,
        out_type=[
            jax.ShapeDtypeStruct((NC * NP,), jnp.float32),
            jax.ShapeDtypeStruct((BKT_WORDS,), jnp.int32),
            jax.ShapeDtypeStruct((NW * 16,), jnp.int32),
        ],
        mesh=plsc.VectorSubcoreMesh(**_MESH),
        compiler_params=pltpu.CompilerParams(needs_layout_passes=False),
        scratch_types=[
            pltpu.VMEM((CH, 128), jnp.int32),
            pltpu.VMEM((CH, 128), jnp.int32),
            pltpu.VMEM((CH, 128), jnp.float32),
            pltpu.VMEM((NPART * PCAP,), jnp.int32),
            pltpu.VMEM((NPART * PCAP,), jnp.int32),
            pltpu.VMEM((NPART * PCAP,), jnp.float32),
            pltpu.VMEM((GREC,), jnp.int32),
            pltpu.VMEM((16,), jnp.int32),
            pltpu.VMEM(((NP // NS),), jnp.float32),
            pltpu.VMEM_SHARED((NP,), jnp.float32),
        ],
    )(colb, rowb, ewb)


def _splat_lane(ewv, j, r):
    jv = jnp.full((16,), j, jnp.int32)
    rv = jnp.broadcast_to(r, (16,)).astype(jnp.int32)
    return plsc.load_gather(ewv, [jv, rv])


def _conv1_body(colb, rowb, ewb, g1, out, colv, rowv, ewv, rbuf, zv,
                slab):
    c = lax.axis_index("c")
    s = lax.axis_index("s")
    w = _worker(c, s)
    zrows = NP // NS

    def zfill(i, _):
        zv[i, :] = jnp.zeros((16,), jnp.float32)
        return 0

    lax.fori_loop(0, zv.shape[0], zfill, 0)
    for k in range(zrows // zv.shape[0]):
        pltpu.sync_copy(zv, slab.at[pl.ds(s * zrows + k * zv.shape[0],
                                          zv.shape[0])])
    plsc.subcore_barrier()

    base = w * BPT

    def body(i, _):
        cb = base + i * CH
        pltpu.sync_copy(colb.at[pl.ds(cb, CH)], colv)
        pltpu.sync_copy(rowb.at[pl.ds(cb, CH)], rowv)
        pltpu.sync_copy(ewb.at[pl.ds(cb, CH)], ewv)
        for j in range(CH):
            pltpu.sync_copy(g1.at[rowv.at[j]], rbuf)

            def scale(r, _):
                rbuf[r, :] = rbuf[r, :] * _splat_lane(ewv, j, r)
                return 0

            lax.fori_loop(0, 128, scale, 0)
            pltpu.sync_copy(rbuf, slab.at[colv.at[j]], add=True)
        return 0

    lax.fori_loop(0, NCHUNK, body, 0)
    plsc.subcore_barrier()
    zr = zv.shape[0]
    for k in range(zrows // zr):
        pltpu.sync_copy(slab.at[pl.ds(s * zrows + k * zr, zr)], zv)
        pltpu.sync_copy(zv, out.at[c, pl.ds(s * zrows + k * zr, zr)])


def _conv1(colb, rowb, ewb, g1):
    return pl.kernel(
        _conv1_body,
        out_type=jax.ShapeDtypeStruct((NC, NP, 16), jnp.float32),
        mesh=plsc.VectorSubcoreMesh(**_MESH),
        compiler_params=pltpu.CompilerParams(use_tc_tiling_on_sc=False,
                                             needs_layout_passes=False),
        scratch_types=[
            pltpu.VMEM((CH, 128), jnp.int32),
            pltpu.VMEM((CH, 128), jnp.int32),
            pltpu.VMEM((CH, 128), jnp.float32),
            pltpu.VMEM((128, 16), jnp.float32),
            pltpu.VMEM((344, 16), jnp.float32),
            pltpu.VMEM_SHARED((NP, 16), jnp.float32),
        ],
    )(colb, rowb, ewb, g1)


def _p2_body(g2, bkt, cnts, out, rec0, rec1, wst, rbuf0, rbuf1, zb,
             cbuf, slab, gs0, gs1, ss0, ss1, ws0, ws1):
    c = lax.axis_index("c")
    s = lax.axis_index("s")
    iota16 = lax.iota(jnp.int32, 16)
    recs = (rec0, rec1)
    rbufs = (rbuf0, rbuf1)
    gsems = (gs0, gs1)
    ssems = (ss0, ss1)
    wsems = (ws0, ws1)

    def zfill(i, _):
        for t in range(8):
            zb[i, pl.ds(t * 16, 16)] = jnp.zeros((16,), jnp.float32)
        return 0

    lax.fori_loop(0, zb.shape[0], zfill, 0)

    def gdesc(par):
        return pltpu.make_async_copy(
            g2.at[recs[par].at[pl.ds(128, 128)]], rbufs[par], gsems[par])

    def sdesc(par, addr):
        return pltpu.make_async_copy(
            bkt.at[pl.ds(addr, GREC)], recs[par], ssems[par])

    def wdesc(par):
        return pltpu.make_async_copy(
            rbufs[par], slab.at[wst.at[par]], wsems[par])

    def scale(par):
        rec = recs[par]
        rbuf = rbufs[par]

        def sbody(r, _):
            pv = jnp.broadcast_to(256 + r, (16,)).astype(jnp.int32)
            e = plsc.bitcast(plsc.load_gather(rec, [pv]), jnp.float32)
            for t in range(8):
                rbuf[r, pl.ds(t * 16, 16)] = rbuf[r, pl.ds(t * 16, 16)] * e
            return 0

        lax.fori_loop(0, 128, sbody, 0)

    for q in range(4):
        p = 4 * c + q
        lo = p * PS
        for k in range(SPT // zb.shape[0]):
            pltpu.sync_copy(zb, slab.at[pl.ds(s * SPT + k * zb.shape[0],
                                              zb.shape[0])])
        plsc.subcore_barrier()
        for reg in range(2):
            t = 2 * s + reg
            rbase = (t * NPART + p) * GCAPW
            pltpu.sync_copy(cnts.at[pl.ds(t * 16, 16)], cbuf)
            ctv = cbuf[pl.ds(0, 16)]
            ng = jnp.sum(jnp.where(iota16 == p, ctv, jnp.zeros((16,),
                                                               jnp.int32)))

            def iter_body(i, _):
                pltpu.sync_copy(bkt.at[pl.ds(rbase + i * GREC, GREC)], rec0)
                pltpu.sync_copy(g2.at[rec0.at[pl.ds(128, 128)]], rbuf0)
                scale(0)
                for tt in range(8):
                    wst[0, pl.ds(tt * 16, 16)] = rec0[pl.ds(tt * 16, 16)]
                pltpu.sync_copy(rbuf0, slab.at[wst.at[0]], add=True)
                return 0

            lax.fori_loop(0, ng, iter_body, 0)
        plsc.subcore_barrier()
        pltpu.sync_copy(slab.at[pl.ds(s * OPT, OPT)],
                        out.at[pl.ds(lo + s * OPT, OPT)])
        plsc.subcore_barrier()


def _p2(g2, bkt, cnts):
    return pl.kernel(
        _p2_body,
        out_type=jax.ShapeDtypeStruct((NP, HID), jnp.float32),
        mesh=plsc.VectorSubcoreMesh(**_MESH),
        compiler_params=pltpu.CompilerParams(needs_layout_passes=False),
        scratch_types=[
            pltpu.VMEM((GREC,), jnp.int32),
            pltpu.VMEM((GREC,), jnp.int32),
            pltpu.VMEM((2, 128), jnp.int32),
            pltpu.VMEM((128, HID), jnp.float32),
            pltpu.VMEM((128, HID), jnp.float32),
            pltpu.VMEM((43, 128), jnp.float32),
            pltpu.VMEM((16,), jnp.int32),
            pltpu.VMEM_SHARED((SLAB, HID), jnp.float32),
        ] + [pltpu.SemaphoreType.DMA] * 6,
    )(g2, bkt, cnts)


_TB = 1024


def _t0_body(degp_ref, xp_ref, g1_ref, dinv_ref):
    d = degp_ref[0, :] + degp_ref[1, :] + 1.0
    dinv = lax.rsqrt(d)[:, None]
    g1_ref[...] = xp_ref[...] * dinv
    dinv_ref[...] = dinv


def _t0(degp, xp):
    return pl.pallas_call(
        _t0_body,
        grid=(NP // _TB,),
        in_specs=[
            pl.BlockSpec((NC, _TB), lambda i: (0, i)),
            pl.BlockSpec((_TB, 16), lambda i: (i, 0)),
        ],
        out_specs=[
            pl.BlockSpec((_TB, 16), lambda i: (i, 0)),
            pl.BlockSpec((_TB, 1), lambda i: (i, 0)),
        ],
        out_shape=[
            jax.ShapeDtypeStruct((NP, 16), jnp.float32),
            jax.ShapeDtypeStruct((NP, 1), jnp.float32),
        ],
    )(degp, xp)


def _t1_body(s1p_ref, xp_ref, dinv_ref, w1_ref, b1_ref, h1_ref, g2_ref):
    dinv = dinv_ref[...]
    y1 = dinv * (s1p_ref[0] + s1p_ref[1]) + dinv * dinv * xp_ref[...]
    h1 = _leaky(jnp.dot(y1, w1_ref[...], preferred_element_type=jnp.float32)
                + b1_ref[...])
    h1_ref[...] = h1
    g2_ref[...] = h1 * dinv


def _t1(s1p, xp, dinv, W1p, b1):
    return pl.pallas_call(
        _t1_body,
        grid=(NP // _TB,),
        in_specs=[
            pl.BlockSpec((NC, _TB, 16), lambda i: (0, i, 0)),
            pl.BlockSpec((_TB, 16), lambda i: (i, 0)),
            pl.BlockSpec((_TB, 1), lambda i: (i, 0)),
            pl.BlockSpec((16, HID), lambda i: (0, 0)),
            pl.BlockSpec((HID,), lambda i: (0,)),
        ],
        out_specs=[
            pl.BlockSpec((_TB, HID), lambda i: (i, 0)),
            pl.BlockSpec((_TB, HID), lambda i: (i, 0)),
        ],
        out_shape=[
            jax.ShapeDtypeStruct((NP, HID), jnp.float32),
            jax.ShapeDtypeStruct((NP, HID), jnp.float32),
        ],
    )(s1p, xp, dinv, W1p, b1)


def _t2_body(s2_ref, h1_ref, dinv_ref, w2_ref, b2_ref, h2_ref):
    dinv = dinv_ref[...]
    y2 = dinv * s2_ref[...] + dinv * dinv * h1_ref[...]
    h2_ref[...] = (jnp.dot(y2, w2_ref[...], preferred_element_type=jnp.float32)
                   + b2_ref[...])


def _t2(s2, h1, dinv, W2, b2):
    return pl.pallas_call(
        _t2_body,
        grid=(NP // _TB,),
        in_specs=[
            pl.BlockSpec((_TB, HID), lambda i: (i, 0)),
            pl.BlockSpec((_TB, HID), lambda i: (i, 0)),
            pl.BlockSpec((_TB, 1), lambda i: (i, 0)),
            pl.BlockSpec((HID, HID), lambda i: (0, 0)),
            pl.BlockSpec((HID,), lambda i: (0,)),
        ],
        out_specs=pl.BlockSpec((_TB, HID), lambda i: (i, 0)),
        out_shape=jax.ShapeDtypeStruct((NP, HID), jnp.float32),
    )(s2, h1, dinv, W2, b2)


def _head_body(flat_ref, wf0_ref, bf0_ref, wf1_ref, bf1_ref, wo_ref, bo_ref,
               out_ref):
    z = _leaky(
        jnp.dot(flat_ref[...], wf0_ref[...], preferred_element_type=jnp.float32)
        + bf0_ref[...]
    )
    z = _leaky(
        jnp.dot(z, wf1_ref[...], preferred_element_type=jnp.float32)
        + bf1_ref[...]
    )
    z = jnp.dot(z, wo_ref[...], preferred_element_type=jnp.float32) + bo_ref[...]
    out_ref[...] = jnp.tanh(z) * 90.0 + 150.0


def _head(flat, Wf0, bf0, Wf1, bf1, Wo, bo):
    B = 400
    return pl.pallas_call(
        _head_body,
        grid=(ROWS // B,),
        in_specs=[
            pl.BlockSpec((B, GROUP * HID), lambda i: (i, 0)),
            pl.BlockSpec((GROUP * HID, HID), lambda i: (0, 0)),
            pl.BlockSpec((HID,), lambda i: (0,)),
            pl.BlockSpec((HID, HID), lambda i: (0, 0)),
            pl.BlockSpec((HID,), lambda i: (0,)),
            pl.BlockSpec((HID, 1), lambda i: (0, 0)),
            pl.BlockSpec((1,), lambda i: (0,)),
        ],
        out_specs=pl.BlockSpec((B, 1), lambda i: (i, 0)),
        out_shape=jax.ShapeDtypeStruct((ROWS, 1), jnp.float32),
    )(flat, Wf0, bf0, Wf1, bf1, Wo, bo)


def kernel(x, edge_index, edge_weight, W1, b1, W2, b2, Wf0, bf0, Wf1, bf1,
           Wo, bo):
    row = edge_index[0]
    col = edge_index[1]
    ew = edge_weight

    pad = EP - E
    rowb = jnp.concatenate([row, jnp.zeros((pad,), row.dtype)]).reshape(
        NBLK, 128)
    colb = jnp.concatenate([col, jnp.full((pad,), NP - 1, col.dtype)]).reshape(
        NBLK, 128)
    ewb = jnp.concatenate([ew, jnp.zeros((pad,), ew.dtype)]).reshape(NBLK, 128)
    xp = jnp.pad(x, ((0, NP - N), (0, 13)))
    W1p = jnp.pad(W1, ((0, 13), (0, 0)))

    degf, bkt, cnts = _p1(colb, rowb, ewb)
    degp = degf.reshape(NC, NP)
    g1, dinv = _t0(degp, xp)
    s1p = _conv1(colb, rowb, ewb, g1)
    h1, g2 = _t1(s1p, xp, dinv, W1p, b1)
    s2 = _p2(g2, bkt, cnts)
    h2 = _t2(s2, h1, dinv, W2, b2)
    flat = h2[:N].reshape(ROWS, GROUP * HID)
    return _head(flat, Wf0, bf0, Wf1, bf1, Wo, bo)


# speedup vs baseline: 12.3785x; 12.3785x over previous
import functools

import jax
import jax.numpy as jnp
from jax import lax
from jax.experimental import pallas as pl
from jax.experimental.pallas import tpu as pltpu
from jax.experimental.pallas import tpu_sc as plsc

N = 44000
E = 704000
HID = 128
GROUP = 22
ROWS = N // GROUP

NP = 44032
NBLK = 5504
EP = NBLK * 128
NC, NS = 2, 16
NW = NC * NS
BPT = NBLK // NW
CH = 4
NCHUNK = BPT // CH

NPART = 8
PS = NP // NPART
SLAB = PS + 16
DUMP = PS
SPT = SLAB // NS
OPT = PS // NS
CCAP = 768

_MESH = dict(core_axis_name="c", subcore_axis_name="s", num_cores=NC,
             num_subcores=NS)


def _leaky(v):
    return jnp.where(v >= 0, v, 0.01 * v)


def _worker(c, s):
    return c * NS + s


def _splat_lane(ewv, j, r):
    jv = jnp.full((16,), j, jnp.int32)
    rv = jnp.broadcast_to(r, (16,)).astype(jnp.int32)
    return plsc.load_gather(ewv, [jv, rv])


def _splat_pos(cew, pos):
    pv = jnp.broadcast_to(pos, (16,)).astype(jnp.int32)
    return plsc.load_gather(cew, [pv])


def _deg_body(colb, ewb, out, colv, ewv, zv, degsp):
    c = lax.axis_index("c")
    s = lax.axis_index("s")
    w = _worker(c, s)
    zw = NP // NS

    def zfill(i, _):
        zv[pl.ds(i * 16, 16)] = jnp.zeros((16,), jnp.float32)
        return 0

    lax.fori_loop(0, zw // 16, zfill, 0)
    pltpu.sync_copy(zv, degsp.at[pl.ds(s * zw, zw)])
    plsc.subcore_barrier()

    base = w * BPT

    def body(i, _):
        cb = base + i * CH
        pltpu.sync_copy(colb.at[pl.ds(cb, CH)], colv)
        pltpu.sync_copy(ewb.at[pl.ds(cb, CH)], ewv)
        for j in range(CH):
            pltpu.sync_copy(ewv.at[j], degsp.at[colv.at[j]], add=True)
        return 0

    lax.fori_loop(0, NCHUNK, body, 0)
    plsc.subcore_barrier()
    pltpu.sync_copy(degsp.at[pl.ds(s * zw, zw)], zv)
    pltpu.sync_copy(zv, out.at[pl.ds(c * NP + s * zw, zw)])


def _deg(colb, ewb):
    zr = (NP // NS) // 16
    return pl.kernel(
        _deg_body,
        out_type=jax.ShapeDtypeStruct((NC * NP,), jnp.float32),
        mesh=plsc.VectorSubcoreMesh(**_MESH),
        scratch_types=[
            pltpu.VMEM((CH, 128), jnp.int32),
            pltpu.VMEM((CH, 128), jnp.float32),
            pltpu.VMEM((zr * 16,), jnp.float32),
            pltpu.VMEM_SHARED((NP,), jnp.float32),
        ],
    )(colb, ewb)


def _conv1_body(colb, rowb, ewb, g1, out, colv, rowv, ewv, rbuf, zv, slab):
    c = lax.axis_index("c")
    s = lax.axis_index("s")
    w = _worker(c, s)
    zrows = NP // NS

    def zfill(i, _):
        zv[i, :] = jnp.zeros((16,), jnp.float32)
        return 0

    lax.fori_loop(0, zv.shape[0], zfill, 0)
    for k in range(zrows // zv.shape[0]):
        pltpu.sync_copy(zv, slab.at[pl.ds(s * zrows + k * zv.shape[0],
                                          zv.shape[0])])
    plsc.subcore_barrier()

    base = w * BPT

    def body(i, _):
        cb = base + i * CH
        pltpu.sync_copy(colb.at[pl.ds(cb, CH)], colv)
        pltpu.sync_copy(rowb.at[pl.ds(cb, CH)], rowv)
        pltpu.sync_copy(ewb.at[pl.ds(cb, CH)], ewv)
        for j in range(CH):
            pltpu.sync_copy(g1.at[rowv.at[j]], rbuf)

            def scale(r, _):
                rbuf[r, :] = rbuf[r, :] * _splat_lane(ewv, j, r)
                return 0

            lax.fori_loop(0, 128, scale, 0)
            pltpu.sync_copy(rbuf, slab.at[colv.at[j]], add=True)
        return 0

    lax.fori_loop(0, NCHUNK, body, 0)
    plsc.subcore_barrier()
    zr = zv.shape[0]
    for k in range(zrows // zr):
        pltpu.sync_copy(slab.at[pl.ds(s * zrows + k * zr, zr)], zv)
        pltpu.sync_copy(zv, out.at[c, pl.ds(s * zrows + k * zr, zr)])


def _conv1(colb, rowb, ewb, g1):
    return pl.kernel(
        _conv1_body,
        out_type=jax.ShapeDtypeStruct((NC, NP, 16), jnp.float32),
        mesh=plsc.VectorSubcoreMesh(**_MESH),
        compiler_params=pltpu.CompilerParams(use_tc_tiling_on_sc=False,
                                             needs_layout_passes=False),
        scratch_types=[
            pltpu.VMEM((CH, 128), jnp.int32),
            pltpu.VMEM((CH, 128), jnp.int32),
            pltpu.VMEM((CH, 128), jnp.float32),
            pltpu.VMEM((128, 16), jnp.float32),
            pltpu.VMEM((344, 16), jnp.float32),
            pltpu.VMEM_SHARED((NP, 16), jnp.float32),
        ],
    )(colb, rowb, ewb, g1)


def _conv2_body(colb, rowb, ewb, g2, out, colv, rowv, ewv, ccol, crow, cew,
                stage, rbuf, obuf, zb, slab):
    c = lax.axis_index("c")
    s = lax.axis_index("s")
    bpt = NBLK // NS
    base = s * bpt
    nchunk = bpt // CH

    def zfill(i, _):
        for t in range(8):
            zb[i, pl.ds(t * 16, 16)] = jnp.zeros((16,), jnp.float32)
        return 0

    lax.fori_loop(0, zb.shape[0], zfill, 0)

    def fire(j, _):
        for t in range(8):
            stage[0, pl.ds(t * 16, 16)] = ccol[pl.ds(j * 128 + t * 16, 16)]
        pltpu.sync_copy(g2.at[crow.at[pl.ds(j * 128, 128)]], rbuf)

        def scale(r, _):
            e = _splat_pos(cew, j * 128 + r)
            for t in range(8):
                rbuf[r, pl.ds(t * 16, 16)] = rbuf[r, pl.ds(t * 16, 16)] * e
            return 0

        lax.fori_loop(0, 128, scale, 0)
        pltpu.sync_copy(rbuf, slab.at[stage.at[0]], add=True)
        return 0

    for q in range(4):
        p = 4 * c + q
        lo = p * PS

        for k in range(SPT // zb.shape[0]):
            pltpu.sync_copy(zb, slab.at[pl.ds(s * SPT + k * zb.shape[0],
                                              zb.shape[0])])
        plsc.subcore_barrier()

        def body(i, cnt):
            cb = base + i * CH
            pltpu.sync_copy(colb.at[pl.ds(cb, CH)], colv)
            pltpu.sync_copy(rowb.at[pl.ds(cb, CH)], rowv)
            pltpu.sync_copy(ewb.at[pl.ds(cb, CH)], ewv)
            masks = []
            pops = []
            for g in range(CH * 8):
                jj, gg = divmod(g, 8)
                c16 = colv[jj, pl.ds(gg * 16, 16)]
                m = (c16 >= lo) & (c16 < lo + PS)
                masks.append(m)
                pops.append(jnp.sum(m.astype(jnp.int32)))
            for g in range(CH * 8):
                jj, gg = divmod(g, 8)
                c16 = colv[jj, pl.ds(gg * 16, 16)]
                r16 = rowv[jj, pl.ds(gg * 16, 16)]
                e16 = ewv[jj, pl.ds(gg * 16, 16)]
                m = masks[g]
                plsc.store_compressed(ccol.at[pl.ds(cnt, 16)], c16 - lo,
                                      mask=m)
                plsc.store_compressed(crow.at[pl.ds(cnt, 16)], r16, mask=m)
                plsc.store_compressed(cew.at[pl.ds(cnt, 16)], e16, mask=m)
                cnt = cnt + pops[g]
            ng = cnt // 128
            lax.fori_loop(0, ng, fire, 0)
            off = ng * 128
            for t in range(8):
                ccol[pl.ds(t * 16, 16)] = ccol[pl.ds(off + t * 16, 16)]
                crow[pl.ds(t * 16, 16)] = crow[pl.ds(off + t * 16, 16)]
                cew[pl.ds(t * 16, 16)] = cew[pl.ds(off + t * 16, 16)]
            return cnt - off

        cnt = lax.fori_loop(0, nchunk, body, jnp.int32(0))
        for t in range(8):
            ccol[pl.ds(cnt + t * 16, 16)] = jnp.full((16,), DUMP, jnp.int32)
            crow[pl.ds(cnt + t * 16, 16)] = jnp.zeros((16,), jnp.int32)
            cew[pl.ds(cnt + t * 16, 16)] = jnp.zeros((16,), jnp.float32)
        lax.fori_loop(0, (cnt + 127) // 128, fire, 0)
        plsc.subcore_barrier()
        pltpu.sync_copy(slab.at[pl.ds(s * OPT, OPT)], obuf)
        pltpu.sync_copy(obuf, out.at[pl.ds(lo + s * OPT, OPT)])
        plsc.subcore_barrier()


def _conv2(colb, rowb, ewb, g2):
    return pl.kernel(
        _conv2_body,
        out_type=jax.ShapeDtypeStruct((NP, HID), jnp.float32),
        mesh=plsc.VectorSubcoreMesh(**_MESH),
        compiler_params=pltpu.CompilerParams(needs_layout_passes=False),
        scratch_types=[
            pltpu.VMEM((CH, 128), jnp.int32),
            pltpu.VMEM((CH, 128), jnp.int32),
            pltpu.VMEM((CH, 128), jnp.float32),
            pltpu.VMEM((CCAP + 144,), jnp.int32),
            pltpu.VMEM((CCAP + 144,), jnp.int32),
            pltpu.VMEM((CCAP + 144,), jnp.float32),
            pltpu.VMEM((1, 128), jnp.int32),
            pltpu.VMEM((128, HID), jnp.float32),
            pltpu.VMEM((OPT, HID), jnp.float32),
            pltpu.VMEM((69, 128), jnp.float32),
            pltpu.VMEM_SHARED((SLAB, HID), jnp.float32),
        ],
    )(colb, rowb, ewb, g2)


_TB = 1024


def _t0_body(degp_ref, xp_ref, g1_ref, dinv_ref):
    d = degp_ref[0, :] + degp_ref[1, :] + 1.0
    dinv = lax.rsqrt(d)[:, None]
    g1_ref[...] = xp_ref[...] * dinv
    dinv_ref[...] = dinv


def _t0(degp, xp):
    return pl.pallas_call(
        _t0_body,
        grid=(NP // _TB,),
        in_specs=[
            pl.BlockSpec((NC, _TB), lambda i: (0, i)),
            pl.BlockSpec((_TB, 16), lambda i: (i, 0)),
        ],
        out_specs=[
            pl.BlockSpec((_TB, 16), lambda i: (i, 0)),
            pl.BlockSpec((_TB, 1), lambda i: (i, 0)),
        ],
        out_shape=[
            jax.ShapeDtypeStruct((NP, 16), jnp.float32),
            jax.ShapeDtypeStruct((NP, 1), jnp.float32),
        ],
    )(degp, xp)


def _t1_body(s1p_ref, xp_ref, dinv_ref, w1_ref, b1_ref, h1_ref, g2_ref):
    dinv = dinv_ref[...]
    y1 = dinv * (s1p_ref[0] + s1p_ref[1]) + dinv * dinv * xp_ref[...]
    h1 = _leaky(jnp.dot(y1, w1_ref[...], preferred_element_type=jnp.float32)
                + b1_ref[...])
    h1_ref[...] = h1
    g2_ref[...] = h1 * dinv


def _t1(s1p, xp, dinv, W1p, b1):
    return pl.pallas_call(
        _t1_body,
        grid=(NP // _TB,),
        in_specs=[
            pl.BlockSpec((NC, _TB, 16), lambda i: (0, i, 0)),
            pl.BlockSpec((_TB, 16), lambda i: (i, 0)),
            pl.BlockSpec((_TB, 1), lambda i: (i, 0)),
            pl.BlockSpec((16, HID), lambda i: (0, 0)),
            pl.BlockSpec((HID,), lambda i: (0,)),
        ],
        out_specs=[
            pl.BlockSpec((_TB, HID), lambda i: (i, 0)),
            pl.BlockSpec((_TB, HID), lambda i: (i, 0)),
        ],
        out_shape=[
            jax.ShapeDtypeStruct((NP, HID), jnp.float32),
            jax.ShapeDtypeStruct((NP, HID), jnp.float32),
        ],
    )(s1p, xp, dinv, W1p, b1)


def _t2_body(s2_ref, h1_ref, dinv_ref, w2_ref, b2_ref, h2_ref):
    dinv = dinv_ref[...]
    y2 = dinv * s2_ref[...] + dinv * dinv * h1_ref[...]
    h2_ref[...] = (jnp.dot(y2, w2_ref[...], preferred_element_type=jnp.float32)
                   + b2_ref[...])


def _t2(s2, h1, dinv, W2, b2):
    return pl.pallas_call(
        _t2_body,
        grid=(NP // _TB,),
        in_specs=[
            pl.BlockSpec((_TB, HID), lambda i: (i, 0)),
            pl.BlockSpec((_TB, HID), lambda i: (i, 0)),
            pl.BlockSpec((_TB, 1), lambda i: (i, 0)),
            pl.BlockSpec((HID, HID), lambda i: (0, 0)),
            pl.BlockSpec((HID,), lambda i: (0,)),
        ],
        out_specs=pl.BlockSpec((_TB, HID), lambda i: (i, 0)),
        out_shape=jax.ShapeDtypeStruct((NP, HID), jnp.float32),
    )(s2, h1, dinv, W2, b2)


def _head_body(flat_ref, wf0_ref, bf0_ref, wf1_ref, bf1_ref, wo_ref, bo_ref,
               out_ref):
    z = _leaky(
        jnp.dot(flat_ref[...], wf0_ref[...], preferred_element_type=jnp.float32)
        + bf0_ref[...]
    )
    z = _leaky(
        jnp.dot(z, wf1_ref[...], preferred_element_type=jnp.float32)
        + bf1_ref[...]
    )
    z = jnp.dot(z, wo_ref[...], preferred_element_type=jnp.float32) + bo_ref[...]
    out_ref[...] = jnp.tanh(z) * 90.0 + 150.0


def _head(flat, Wf0, bf0, Wf1, bf1, Wo, bo):
    B = 400
    return pl.pallas_call(
        _head_body,
        grid=(ROWS // B,),
        in_specs=[
            pl.BlockSpec((B, GROUP * HID), lambda i: (i, 0)),
            pl.BlockSpec((GROUP * HID, HID), lambda i: (0, 0)),
            pl.BlockSpec((HID,), lambda i: (0,)),
            pl.BlockSpec((HID, HID), lambda i: (0, 0)),
            pl.BlockSpec((HID,), lambda i: (0,)),
            pl.BlockSpec((HID, 1), lambda i: (0, 0)),
            pl.BlockSpec((1,), lambda i: (0,)),
        ],
        out_specs=pl.BlockSpec((B, 1), lambda i: (i, 0)),
        out_shape=jax.ShapeDtypeStruct((ROWS, 1), jnp.float32),
    )(flat, Wf0, bf0, Wf1, bf1, Wo, bo)


def kernel(x, edge_index, edge_weight, W1, b1, W2, b2, Wf0, bf0, Wf1, bf1,
           Wo, bo):
    row = edge_index[0]
    col = edge_index[1]
    ew = edge_weight

    pad = EP - E
    rowb = jnp.concatenate([row, jnp.zeros((pad,), row.dtype)]).reshape(
        NBLK, 128)
    colb = jnp.concatenate([col, jnp.full((pad,), NP - 1, col.dtype)]).reshape(
        NBLK, 128)
    ewb = jnp.concatenate([ew, jnp.zeros((pad,), ew.dtype)]).reshape(NBLK, 128)
    xp = jnp.pad(x, ((0, NP - N), (0, 13)))
    W1p = jnp.pad(W1, ((0, 13), (0, 0)))

    degp = _deg(colb, ewb).reshape(NC, NP)
    g1, dinv = _t0(degp, xp)
    s1p = _conv1(colb, rowb, ewb, g1)
    h1, g2 = _t1(s1p, xp, dinv, W1p, b1)
    s2 = _conv2(colb, rowb, ewb, g2)
    h2 = _t2(s2, h1, dinv, W2, b2)
    flat = h2[:N].reshape(ROWS, GROUP * HID)
    return _head(flat, Wf0, bf0, Wf1, bf1, Wo, bo)
